# baseline (speedup 1.0000x reference)
import jax
import jax.numpy as jnp
from jax.experimental import pallas as pl

N = 50176
E = 802816
D = 32
NUM_ITER = 70


def _finalize_body(pooled_ref, cnt_ref, out_ref):
    out_ref[...] = jnp.where(cnt_ref[...] > 0.0, pooled_ref[...], 0.0)


def _lp(src, dst, num_iter=NUM_ITER):
    labels0 = jnp.arange(N, dtype=src.dtype)
    ones = jnp.ones(E, dtype=src.dtype)

    def _step(labels):
        nbr = labels[src]
        key = dst * N + nbr
        order = jnp.argsort(key)
        s_key = key[order]
        s_dst = dst[order]
        s_nbr = nbr[order]
        new_run = jnp.concatenate([jnp.array([True]), s_key[1:] != s_key[:-1]])
        run_id = jnp.cumsum(new_run) - 1
        counts = jax.ops.segment_sum(ones, run_id, num_segments=E)
        comp = counts[run_id] * N + s_nbr
        best = jax.ops.segment_max(comp, s_dst, num_segments=N)
        minval = jnp.iinfo(best.dtype).min
        return jnp.where(best == minval, labels, best % N)

    def _cond(state):
        labels, i, done = state
        return jnp.logical_and(i < num_iter, jnp.logical_not(done))

    def _body(state):
        labels, i, done = state
        new_labels = _step(labels)
        return (new_labels, i + 1, jnp.array_equal(new_labels, labels))

    labels, _, _ = jax.lax.while_loop(
        _cond, _body, (labels0, jnp.int32(0), jnp.array(False))
    )
    return labels


def kernel(x, edge_index):
    row, col = edge_index[0], edge_index[1]
    aff = jnp.linalg.norm(x[row] - x[col], axis=1)
    deg = jax.ops.segment_sum(jnp.ones(E, jnp.float32), row, num_segments=N)
    mean_aff = jax.ops.segment_sum(aff, row, num_segments=N) / jnp.maximum(deg, 1.0)
    thresh = jnp.minimum(mean_aff[row], mean_aff[col])
    keep = aff <= thresh
    src_f = jnp.where(keep, row, col)
    dst_f = col
    node_labels = _lp(src_f, dst_f)
    _, inv = jnp.unique(node_labels, return_inverse=True, size=N)
    inv = inv.reshape(-1)
    pooled = jax.ops.segment_max(x, inv, num_segments=N)
    cnt = jax.ops.segment_sum(jnp.ones(N, jnp.float32), inv, num_segments=N)
    cnt_b = jnp.broadcast_to(cnt[:, None], (N, D))
    blk = 1568
    out = pl.pallas_call(
        _finalize_body,
        grid=(N // blk,),
        in_specs=[
            pl.BlockSpec((blk, D), lambda i: (i, jnp.int32(0))),
            pl.BlockSpec((blk, D), lambda i: (i, jnp.int32(0))),
        ],
        out_specs=pl.BlockSpec((blk, D), lambda i: (i, jnp.int32(0))),
        out_shape=jax.ShapeDtypeStruct((N, D), jnp.float32),
    )(pooled, cnt_b)
    return out


# speedup vs baseline: 7.2422x; 7.2422x over previous
import functools

import jax
import jax.numpy as jnp
from jax import lax
from jax.experimental import pallas as pl
from jax.experimental.pallas import tpu as pltpu
from jax.experimental.pallas import tpu_sc as plsc

N = 50176
E = 802816
D = 32
NUM_ITER = 70

NW = 32
PER = N // NW
SENT = 1024
NE = N + SENT
GCAP = 4096
CAP = 57344
SZ = E + 16 * N + CAP

_mesh = plsc.VectorSubcoreMesh(core_axis_name="c", subcore_axis_name="s")


@functools.partial(
    pl.kernel,
    out_type=jax.ShapeDtypeStruct((N,), jnp.int32),
    mesh=_mesh,
    compiler_params=pltpu.CompilerParams(needs_layout_passes=False),
    scratch_types=[
        pltpu.VMEM((NE,), jnp.int32),
        pltpu.VMEM((CAP,), jnp.int32),
        pltpu.VMEM((PER + 24,), jnp.int32),
        pltpu.VMEM((GCAP,), jnp.int32),
        pltpu.VMEM((PER,), jnp.int32),
    ],
)
def _lp_step(lab_hbm, src_hbm, voff_hbm, out_hbm, lab_v, src_v, voff_v, g_v, out_v):
    c = lax.axis_index("c")
    s = lax.axis_index("s")
    wid = s * 2 + c
    nbase = wid * PER
    pltpu.sync_copy(lab_hbm, lab_v)
    pltpu.sync_copy(voff_hbm.at[pl.ds(nbase, PER + 8)], voff_v.at[pl.ds(0, PER + 8)])
    voff0 = voff_v[pl.ds(0, 16)][0]
    pltpu.sync_copy(src_hbm.at[pl.ds(voff0 * 16, CAP)], src_v)

    iota16 = lax.iota(jnp.int32, 16)
    rots = [lax.rem(iota16 + jnp.int32(r), jnp.int32(16)) for r in range(16)]
    lane0 = iota16 == 0

    def node_body(d, carry):
        vpair = voff_v[pl.ds(d, 16)]
        v0 = vpair[0] - voff0
        nv = jnp.minimum(vpair[1] - vpair[0], GCAP // 16)
        old = lab_v[pl.ds(nbase + d, 16)][0]

        def gather_one(i, carry2):
            sidx = src_v[pl.ds((v0 + i) * 16, 16)]
            g_v[pl.ds(i * 16, 16)] = plsc.load_gather(lab_v, [sidx])
            return carry2

        lax.fori_loop(jnp.int32(0), nv, gather_one, jnp.int32(0))

        def best_i(i, best):
            gi = g_v[pl.ds(i * 16, 16)]

            def cnt_j(j, cnt):
                gj = g_v[pl.ds(j * 16, 16)]
                for r in range(16):
                    gr = gj.at[rots[r]].get(mode="promise_in_bounds")
                    cnt = cnt + (gi == gr).astype(jnp.int32)
                return cnt

            cnt = lax.fori_loop(jnp.int32(0), nv, cnt_j, jnp.zeros(16, jnp.int32))
            valid = gi < jnp.int32(0x40000000)
            comp = jnp.where(
                valid,
                jnp.left_shift(jnp.minimum(cnt, jnp.int32(0x7FFF)), 16) | gi,
                jnp.int32(0),
            )
            return jnp.maximum(best, jnp.max(comp))

        best = lax.fori_loop(jnp.int32(0), nv, best_i, jnp.int32(0))
        new = jnp.where(best > 0, best & jnp.int32(0xFFFF), old)
        plsc.store_scatter(
            out_v,
            [jnp.broadcast_to(d, (16,))],
            jnp.broadcast_to(new, (16,)),
            mask=lane0,
        )
        return carry

    lax.fori_loop(jnp.int32(0), jnp.int32(PER), node_body, jnp.int32(0))
    pltpu.sync_copy(out_v, out_hbm.at[pl.ds(nbase, PER)])


def _build_csr(src_i, dst_i):
    order = jnp.argsort(dst_i)
    s_dst = dst_i[order]
    s_src = src_i[order]
    iota_e = jnp.arange(E, dtype=jnp.int32)
    run_start = jnp.concatenate(
        [jnp.ones((1,), jnp.bool_), s_dst[1:] != s_dst[:-1]]
    )
    seg_start = lax.cummax(jnp.where(run_start, iota_e, 0))
    rank = iota_e - seg_start
    ptr = jnp.searchsorted(s_dst, jnp.arange(N + 1, dtype=jnp.int32)).astype(jnp.int32)
    indeg = ptr[1:] - ptr[:-1]
    nv = (indeg + 15) // 16
    voff_n = jnp.concatenate(
        [jnp.zeros((1,), jnp.int32), jnp.cumsum(nv, dtype=jnp.int32)]
    )
    voff = jnp.concatenate([voff_n, jnp.broadcast_to(voff_n[N], (7,))])
    dest = jnp.take(voff_n, s_dst) * 16 + rank
    pad_init = jnp.int32(N) + (jnp.arange(SZ, dtype=jnp.int32) & 1023)
    src_pad = pad_init.at[dest].add(s_src - (jnp.int32(N) + (dest & 1023)))
    return src_pad, voff


def kernel(x, edge_index):
    row, col = edge_index[0], edge_index[1]
    aff = jnp.linalg.norm(x[row] - x[col], axis=1)
    deg = jax.ops.segment_sum(jnp.ones(E, jnp.float32), row, num_segments=N)
    mean_aff = jax.ops.segment_sum(aff, row, num_segments=N) / jnp.maximum(deg, 1.0)
    thresh = jnp.minimum(mean_aff[row], mean_aff[col])
    keep = aff <= thresh
    src_i = jnp.where(keep, row, col).astype(jnp.int32)
    dst_i = col.astype(jnp.int32)

    src_pad, voff = _build_csr(src_i, dst_i)

    sent_vals = jnp.int32(0x40000000) + jnp.arange(SENT, dtype=jnp.int32)

    def cond(st):
        _, i, done = st
        return jnp.logical_and(i < NUM_ITER, jnp.logical_not(done))

    def body(st):
        lab, i, _ = st
        ext = jnp.concatenate([lab, sent_vals])
        new = _lp_step(ext, src_pad, voff)
        return new, i + 1, jnp.array_equal(new, lab)

    labels, _, _ = lax.while_loop(
        cond,
        body,
        (jnp.arange(N, dtype=jnp.int32), jnp.int32(0), jnp.array(False)),
    )

    present = jnp.zeros((N,), jnp.int32).at[labels].add(1)
    rank_incl = jnp.cumsum((present > 0).astype(jnp.int32), dtype=jnp.int32)
    inv = jnp.take(rank_incl, labels) - 1
    num_clusters = rank_incl[N - 1]

    pooled = jax.ops.segment_max(x, inv, num_segments=N)
    nonempty = jnp.arange(N, dtype=jnp.int32) < num_clusters
    return jnp.where(nonempty[:, None], pooled, 0.0)


# speedup vs baseline: 14.6001x; 2.0160x over previous
import functools

import jax
import jax.numpy as jnp
from jax import lax
from jax.experimental import pallas as pl
from jax.experimental.pallas import tpu as pltpu
from jax.experimental.pallas import tpu_sc as plsc

N = 50176
E = 802816
D = 32
NUM_ITER = 70

NW = 32
PER = N // NW
SENT = 1024
NE = N + SENT
GCAP = 4096
CAP = 57344
SZ = E + 16 * N + CAP

_mesh = plsc.VectorSubcoreMesh(core_axis_name="c", subcore_axis_name="s")


@functools.partial(
    pl.kernel,
    out_type=jax.ShapeDtypeStruct((N,), jnp.int32),
    mesh=_mesh,
    compiler_params=pltpu.CompilerParams(needs_layout_passes=False),
    scratch_types=[
        pltpu.VMEM((NE,), jnp.int32),
        pltpu.VMEM((CAP,), jnp.int32),
        pltpu.VMEM((PER + 24,), jnp.int32),
        pltpu.VMEM((GCAP,), jnp.int32),
        pltpu.VMEM((PER,), jnp.int32),
    ],
)
def _lp_step(lab_hbm, src_hbm, voff_hbm, out_hbm, lab_v, src_v, voff_v, g_v, out_v):
    c = lax.axis_index("c")
    s = lax.axis_index("s")
    wid = s * 2 + c
    nbase = wid * PER
    pltpu.sync_copy(lab_hbm, lab_v)
    pltpu.sync_copy(voff_hbm.at[pl.ds(nbase, PER + 8)], voff_v.at[pl.ds(0, PER + 8)])
    voff0 = voff_v[pl.ds(0, 16)][0]
    pltpu.sync_copy(src_hbm.at[pl.ds(voff0 * 16, CAP)], src_v)

    iota16 = lax.iota(jnp.int32, 16)
    rots = [lax.rem(iota16 + jnp.int32(r), jnp.int32(16)) for r in range(16)]
    lane0 = iota16 == 0

    def node_body(d, carry):
        vpair = voff_v[pl.ds(d, 16)]
        v0 = vpair[0] - voff0
        nv = jnp.minimum(vpair[1] - vpair[0], GCAP // 16)
        old = lab_v[pl.ds(nbase + d, 16)][0]

        def gather_one(i, carry2):
            sidx = src_v[pl.ds((v0 + i) * 16, 16)]
            g_v[pl.ds(i * 16, 16)] = plsc.load_gather(lab_v, [sidx])
            return carry2

        lax.fori_loop(jnp.int32(0), nv, gather_one, jnp.int32(0))

        def best_i(i, best):
            gi = g_v[pl.ds(i * 16, 16)]

            def cnt_j(j, cnt):
                gj = g_v[pl.ds(j * 16, 16)]
                for r in range(16):
                    gr = gj.at[rots[r]].get(mode="promise_in_bounds")
                    cnt = cnt + (gi == gr).astype(jnp.int32)
                return cnt

            cnt = lax.fori_loop(jnp.int32(0), nv, cnt_j, jnp.zeros(16, jnp.int32))
            valid = gi < jnp.int32(0x40000000)
            comp = jnp.where(
                valid,
                jnp.left_shift(jnp.minimum(cnt, jnp.int32(0x7FFF)), 16) | gi,
                jnp.int32(0),
            )
            return jnp.maximum(best, jnp.max(comp))

        best = lax.fori_loop(jnp.int32(0), nv, best_i, jnp.int32(0))
        new = jnp.where(best > 0, best & jnp.int32(0xFFFF), old)
        plsc.store_scatter(
            out_v,
            [jnp.broadcast_to(d, (16,))],
            jnp.broadcast_to(new, (16,)),
            mask=lane0,
        )
        return carry

    lax.fori_loop(jnp.int32(0), jnp.int32(PER), node_body, jnp.int32(0))
    pltpu.sync_copy(out_v, out_hbm.at[pl.ds(nbase, PER)])


def _build_csr(src_i, dst_i):
    order = jnp.argsort(dst_i)
    s_dst = dst_i[order]
    s_src = src_i[order]
    iota_e = jnp.arange(E, dtype=jnp.int32)
    run_start = jnp.concatenate(
        [jnp.ones((1,), jnp.bool_), s_dst[1:] != s_dst[:-1]]
    )
    seg_start = lax.cummax(jnp.where(run_start, iota_e, 0))
    rank = iota_e - seg_start
    indeg = jnp.zeros((N,), jnp.int32).at[dst_i].add(1)
    nv = (indeg + 15) // 16
    voff_n = jnp.concatenate(
        [jnp.zeros((1,), jnp.int32), jnp.cumsum(nv, dtype=jnp.int32)]
    )
    voff = jnp.concatenate([voff_n, jnp.broadcast_to(voff_n[N], (7,))])
    dest = jnp.take(voff_n, s_dst) * 16 + rank
    pad_init = jnp.int32(N) + (jnp.arange(SZ, dtype=jnp.int32) & 1023)
    src_pad = pad_init.at[dest].add(s_src - (jnp.int32(N) + (dest & 1023)))
    return src_pad, voff


def kernel(x, edge_index):
    row, col = edge_index[0], edge_index[1]
    aff = jnp.linalg.norm(x[row] - x[col], axis=1)
    deg = jax.ops.segment_sum(jnp.ones(E, jnp.float32), row, num_segments=N)
    mean_aff = jax.ops.segment_sum(aff, row, num_segments=N) / jnp.maximum(deg, 1.0)
    thresh = jnp.minimum(mean_aff[row], mean_aff[col])
    keep = aff <= thresh
    src_i = jnp.where(keep, row, col).astype(jnp.int32)
    dst_i = col.astype(jnp.int32)

    src_pad, voff = _build_csr(src_i, dst_i)

    sent_vals = jnp.int32(0x40000000) + jnp.arange(SENT, dtype=jnp.int32)

    def cond(st):
        _, i, done = st
        return jnp.logical_and(i < NUM_ITER, jnp.logical_not(done))

    def body(st):
        lab, i, _ = st
        ext = jnp.concatenate([lab, sent_vals])
        new = _lp_step(ext, src_pad, voff)
        return new, i + 1, jnp.array_equal(new, lab)

    labels, _, _ = lax.while_loop(
        cond,
        body,
        (jnp.arange(N, dtype=jnp.int32), jnp.int32(0), jnp.array(False)),
    )

    present = jnp.zeros((N,), jnp.int32).at[labels].add(1)
    rank_incl = jnp.cumsum((present > 0).astype(jnp.int32), dtype=jnp.int32)
    inv = jnp.take(rank_incl, labels) - 1
    num_clusters = rank_incl[N - 1]

    pooled = jax.ops.segment_max(x, inv, num_segments=N)
    nonempty = jnp.arange(N, dtype=jnp.int32) < num_clusters
    return jnp.where(nonempty[:, None], pooled, 0.0)


# speedup vs baseline: 41.3830x; 2.8344x over previous
import functools

import jax
import jax.numpy as jnp
from jax import lax
from jax.experimental import pallas as pl
from jax.experimental.pallas import tpu as pltpu
from jax.experimental.pallas import tpu_sc as plsc

N = 50176
E = 802816
D = 32
NUM_ITER = 70

NW = 32
PER = N // NW
SENT = 1024
NE = N + SENT
GCAP = 4096
CAP = 57344
SZ = E + 16 * N + CAP

_mesh = plsc.VectorSubcoreMesh(core_axis_name="c", subcore_axis_name="s")


@functools.partial(
    pl.kernel,
    out_type=jax.ShapeDtypeStruct((N,), jnp.int32),
    mesh=_mesh,
    compiler_params=pltpu.CompilerParams(needs_layout_passes=False),
    scratch_types=[
        pltpu.VMEM((NE,), jnp.int32),
        pltpu.VMEM((CAP,), jnp.int32),
        pltpu.VMEM((PER + 24,), jnp.int32),
        pltpu.VMEM((GCAP,), jnp.int32),
        pltpu.VMEM((PER,), jnp.int32),
    ],
)
def _lp_step(lab_hbm, src_hbm, voff_hbm, out_hbm, lab_v, src_v, voff_v, g_v, out_v):
    c = lax.axis_index("c")
    s = lax.axis_index("s")
    wid = s * 2 + c
    nbase = wid * PER
    pltpu.sync_copy(lab_hbm, lab_v)
    pltpu.sync_copy(voff_hbm.at[pl.ds(nbase, PER + 8)], voff_v.at[pl.ds(0, PER + 8)])
    voff0 = voff_v[pl.ds(0, 16)][0]
    pltpu.sync_copy(src_hbm.at[pl.ds(voff0 * 16, CAP)], src_v)

    iota16 = lax.iota(jnp.int32, 16)
    rots = [lax.rem(iota16 + jnp.int32(r), jnp.int32(16)) for r in range(16)]
    lane0 = iota16 == 0

    def node_body(d, carry):
        vpair = voff_v[pl.ds(d, 16)]
        v0 = vpair[0] - voff0
        nv = jnp.minimum(vpair[1] - vpair[0], GCAP // 16)
        old = lab_v[pl.ds(nbase + d, 16)][0]

        def gather_one(i, carry2):
            sidx = src_v[pl.ds((v0 + i) * 16, 16)]
            g_v[pl.ds(i * 16, 16)] = plsc.load_gather(lab_v, [sidx])
            return carry2

        lax.fori_loop(jnp.int32(0), nv, gather_one, jnp.int32(0))

        def best_i(i, best):
            gi = g_v[pl.ds(i * 16, 16)]

            def cnt_j(j, cnt):
                gj = g_v[pl.ds(j * 16, 16)]
                for r in range(16):
                    gr = gj.at[rots[r]].get(mode="promise_in_bounds")
                    cnt = cnt + (gi == gr).astype(jnp.int32)
                return cnt

            cnt = lax.fori_loop(jnp.int32(0), nv, cnt_j, jnp.zeros(16, jnp.int32))
            valid = gi < jnp.int32(0x40000000)
            comp = jnp.where(
                valid,
                jnp.left_shift(jnp.minimum(cnt, jnp.int32(0x7FFF)), 16) | gi,
                jnp.int32(0),
            )
            return jnp.maximum(best, jnp.max(comp))

        best = lax.fori_loop(jnp.int32(0), nv, best_i, jnp.int32(0))
        new = jnp.where(best > 0, best & jnp.int32(0xFFFF), old)
        plsc.store_scatter(
            out_v,
            [jnp.broadcast_to(d, (16,))],
            jnp.broadcast_to(new, (16,)),
            mask=lane0,
        )
        return carry

    lax.fori_loop(jnp.int32(0), jnp.int32(PER), node_body, jnp.int32(0))
    pltpu.sync_copy(out_v, out_hbm.at[pl.ds(nbase, PER)])


EPW = E // NW
GCH = 12544
SCH = 6272


@functools.partial(
    pl.kernel,
    out_type=jax.ShapeDtypeStruct((D, E), jnp.float32),
    mesh=_mesh,
    compiler_params=pltpu.CompilerParams(needs_layout_passes=False),
    scratch_types=[
        pltpu.VMEM((N,), jnp.float32),
        pltpu.VMEM((GCH,), jnp.int32),
        pltpu.VMEM((GCH,), jnp.int32),
        pltpu.VMEM((GCH,), jnp.float32),
    ],
)
def _diff_gather(xt_hbm, row_hbm, col_hbm, dt_hbm, tab_v, row_v, col_v, out_v):
    c = lax.axis_index("c")
    s = lax.axis_index("s")
    f = s * 2 + c
    pltpu.sync_copy(xt_hbm.at[f], tab_v)

    def chunk(k, carry):
        off = k * GCH
        pltpu.sync_copy(row_hbm.at[pl.ds(off, GCH)], row_v)
        pltpu.sync_copy(col_hbm.at[pl.ds(off, GCH)], col_v)

        def grp(g, carry2):
            ds = pl.ds(g * 16, 16)
            a = plsc.load_gather(tab_v, [row_v[ds]])
            b = plsc.load_gather(tab_v, [col_v[ds]])
            out_v[ds] = a - b
            return carry2

        lax.fori_loop(jnp.int32(0), jnp.int32(GCH // 16), grp, jnp.int32(0))
        pltpu.sync_copy(out_v, dt_hbm.at[f, pl.ds(off, GCH)])
        return carry

    lax.fori_loop(jnp.int32(0), jnp.int32(E // GCH), chunk, jnp.int32(0))


@functools.partial(
    pl.kernel,
    out_type=jax.ShapeDtypeStruct((E,), jnp.int32),
    mesh=_mesh,
    compiler_params=pltpu.CompilerParams(needs_layout_passes=False),
    scratch_types=[
        pltpu.VMEM((N,), jnp.float32),
        pltpu.VMEM((SCH,), jnp.float32),
        pltpu.VMEM((SCH,), jnp.int32),
        pltpu.VMEM((SCH,), jnp.int32),
        pltpu.VMEM((SCH,), jnp.int32),
    ],
)
def _edge_select(ma_hbm, aff_hbm, row_hbm, col_hbm, out_hbm, ma_v, aff_v, row_v, col_v, out_v):
    c = lax.axis_index("c")
    s = lax.axis_index("s")
    base = (s * 2 + c) * EPW
    pltpu.sync_copy(ma_hbm, ma_v)

    def chunk(k, carry):
        off = base + k * SCH
        pltpu.sync_copy(aff_hbm.at[pl.ds(off, SCH)], aff_v)
        pltpu.sync_copy(row_hbm.at[pl.ds(off, SCH)], row_v)
        pltpu.sync_copy(col_hbm.at[pl.ds(off, SCH)], col_v)

        def grp(g, carry2):
            ds = pl.ds(g * 16, 16)
            rv = row_v[ds]
            cv = col_v[ds]
            mr = plsc.load_gather(ma_v, [rv])
            mc = plsc.load_gather(ma_v, [cv])
            av = aff_v[ds]
            out_v[ds] = jnp.where(av <= jnp.minimum(mr, mc), rv, cv)
            return carry2

        lax.fori_loop(jnp.int32(0), jnp.int32(SCH // 16), grp, jnp.int32(0))
        pltpu.sync_copy(out_v, out_hbm.at[pl.ds(off, SCH)])
        return carry

    lax.fori_loop(jnp.int32(0), jnp.int32(EPW // SCH), chunk, jnp.int32(0))


def _build_csr(src_i, dst_i):
    order = jnp.argsort(dst_i)
    s_dst = dst_i[order]
    s_src = src_i[order]
    iota_e = jnp.arange(E, dtype=jnp.int32)
    run_start = jnp.concatenate(
        [jnp.ones((1,), jnp.bool_), s_dst[1:] != s_dst[:-1]]
    )
    seg_start = lax.cummax(jnp.where(run_start, iota_e, 0))
    rank = iota_e - seg_start
    indeg = jnp.zeros((N,), jnp.int32).at[dst_i].add(1)
    nv = (indeg + 15) // 16
    voff_n = jnp.concatenate(
        [jnp.zeros((1,), jnp.int32), jnp.cumsum(nv, dtype=jnp.int32)]
    )
    voff = jnp.concatenate([voff_n, jnp.broadcast_to(voff_n[N], (7,))])
    dest = jnp.take(voff_n, s_dst) * 16 + rank
    pad_init = jnp.int32(N) + (jnp.arange(SZ, dtype=jnp.int32) & 1023)
    src_pad = pad_init.at[dest].add(s_src - (jnp.int32(N) + (dest & 1023)))
    return src_pad, voff


def kernel(x, edge_index):
    row, col = edge_index[0], edge_index[1]
    row32 = row.astype(jnp.int32)
    col32 = col.astype(jnp.int32)
    dt = _diff_gather(x.T, row32, col32)
    aff = jnp.linalg.norm(dt.T, axis=1)
    deg = jax.ops.segment_sum(jnp.ones(E, jnp.float32), row, num_segments=N)
    mean_aff = jax.ops.segment_sum(aff, row, num_segments=N) / jnp.maximum(deg, 1.0)
    src_i = _edge_select(mean_aff, aff, row32, col32)
    dst_i = col32

    src_pad, voff = _build_csr(src_i, dst_i)

    sent_vals = jnp.int32(0x40000000) + jnp.arange(SENT, dtype=jnp.int32)

    def cond(st):
        _, i, done = st
        return jnp.logical_and(i < NUM_ITER, jnp.logical_not(done))

    def body(st):
        lab, i, _ = st
        ext = jnp.concatenate([lab, sent_vals])
        new = _lp_step(ext, src_pad, voff)
        return new, i + 1, jnp.array_equal(new, lab)

    labels, _, _ = lax.while_loop(
        cond,
        body,
        (jnp.arange(N, dtype=jnp.int32), jnp.int32(0), jnp.array(False)),
    )

    present = jnp.zeros((N,), jnp.int32).at[labels].add(1)
    rank_incl = jnp.cumsum((present > 0).astype(jnp.int32), dtype=jnp.int32)
    inv = jnp.take(rank_incl, labels) - 1
    num_clusters = rank_incl[N - 1]

    pooled = jax.ops.segment_max(x, inv, num_segments=N)
    nonempty = jnp.arange(N, dtype=jnp.int32) < num_clusters
    return jnp.where(nonempty[:, None], pooled, 0.0)


# speedup vs baseline: 133.3996x; 3.2235x over previous
import functools

import jax
import jax.numpy as jnp
from jax import lax
from jax.experimental import pallas as pl
from jax.experimental.pallas import tpu as pltpu
from jax.experimental.pallas import tpu_sc as plsc

N = 50176
E = 802816
D = 32
NUM_ITER = 70

NW = 32
PER = N // NW
SENT = 1024
NE = N + SENT
GCAP = 4096
CAP = 57344
SZ = E + 16 * N + CAP

_mesh = plsc.VectorSubcoreMesh(core_axis_name="c", subcore_axis_name="s")


@functools.partial(
    pl.kernel,
    out_type=jax.ShapeDtypeStruct((N,), jnp.int32),
    mesh=_mesh,
    compiler_params=pltpu.CompilerParams(needs_layout_passes=False),
    scratch_types=[
        pltpu.VMEM((NE,), jnp.int32),
        pltpu.VMEM((CAP,), jnp.int32),
        pltpu.VMEM((PER + 24,), jnp.int32),
        pltpu.VMEM((GCAP,), jnp.int32),
        pltpu.VMEM((PER,), jnp.int32),
    ],
)
def _lp_step(lab_hbm, src_hbm, voff_hbm, out_hbm, lab_v, src_v, voff_v, g_v, out_v):
    c = lax.axis_index("c")
    s = lax.axis_index("s")
    wid = s * 2 + c
    nbase = wid * PER
    pltpu.sync_copy(lab_hbm, lab_v)
    pltpu.sync_copy(voff_hbm.at[pl.ds(nbase, PER + 8)], voff_v.at[pl.ds(0, PER + 8)])
    voff0 = voff_v[pl.ds(0, 16)][0]
    pltpu.sync_copy(src_hbm.at[pl.ds(voff0 * 16, CAP)], src_v)

    iota16 = lax.iota(jnp.int32, 16)
    rots = [lax.rem(iota16 + jnp.int32(r), jnp.int32(16)) for r in range(16)]
    lane0 = iota16 == 0

    def node_body(d, carry):
        vpair = voff_v[pl.ds(d, 16)]
        v0 = vpair[0] - voff0
        nv = jnp.minimum(vpair[1] - vpair[0], GCAP // 16)
        old = lab_v[pl.ds(nbase + d, 16)][0]

        def gather_one(i, carry2):
            sidx = src_v[pl.ds((v0 + i) * 16, 16)]
            g_v[pl.ds(i * 16, 16)] = plsc.load_gather(lab_v, [sidx])
            return carry2

        lax.fori_loop(jnp.int32(0), nv, gather_one, jnp.int32(0))

        def best_i(i, best):
            gi = g_v[pl.ds(i * 16, 16)]

            def cnt_j(j, cnt):
                gj = g_v[pl.ds(j * 16, 16)]
                for r in range(16):
                    gr = gj.at[rots[r]].get(mode="promise_in_bounds")
                    cnt = cnt + (gi == gr).astype(jnp.int32)
                return cnt

            cnt = lax.fori_loop(jnp.int32(0), nv, cnt_j, jnp.zeros(16, jnp.int32))
            valid = gi < jnp.int32(0x40000000)
            comp = jnp.where(
                valid,
                jnp.left_shift(jnp.minimum(cnt, jnp.int32(0x7FFF)), 16) | gi,
                jnp.int32(0),
            )
            return jnp.maximum(best, jnp.max(comp))

        best = lax.fori_loop(jnp.int32(0), nv, best_i, jnp.int32(0))
        new = jnp.where(best > 0, best & jnp.int32(0xFFFF), old)
        plsc.store_scatter(
            out_v,
            [jnp.broadcast_to(d, (16,))],
            jnp.broadcast_to(new, (16,)),
            mask=lane0,
        )
        return carry

    lax.fori_loop(jnp.int32(0), jnp.int32(PER), node_body, jnp.int32(0))
    pltpu.sync_copy(out_v, out_hbm.at[pl.ds(nbase, PER)])


EPW = E // NW
GCH = 12544
SCH = 6272


@functools.partial(
    pl.kernel,
    out_type=jax.ShapeDtypeStruct((D, E), jnp.float32),
    mesh=_mesh,
    compiler_params=pltpu.CompilerParams(needs_layout_passes=False),
    scratch_types=[
        pltpu.VMEM((N,), jnp.float32),
        pltpu.VMEM((GCH,), jnp.int32),
        pltpu.VMEM((GCH,), jnp.int32),
        pltpu.VMEM((GCH,), jnp.float32),
    ],
)
def _diff_gather(xt_hbm, row_hbm, col_hbm, dt_hbm, tab_v, row_v, col_v, out_v):
    c = lax.axis_index("c")
    s = lax.axis_index("s")
    f = s * 2 + c
    pltpu.sync_copy(xt_hbm.at[f], tab_v)

    def chunk(k, carry):
        off = k * GCH
        pltpu.sync_copy(row_hbm.at[pl.ds(off, GCH)], row_v)
        pltpu.sync_copy(col_hbm.at[pl.ds(off, GCH)], col_v)

        def grp(g, carry2):
            ds = pl.ds(g * 16, 16)
            a = plsc.load_gather(tab_v, [row_v[ds]])
            b = plsc.load_gather(tab_v, [col_v[ds]])
            out_v[ds] = a - b
            return carry2

        lax.fori_loop(jnp.int32(0), jnp.int32(GCH // 16), grp, jnp.int32(0))
        pltpu.sync_copy(out_v, dt_hbm.at[f, pl.ds(off, GCH)])
        return carry

    lax.fori_loop(jnp.int32(0), jnp.int32(E // GCH), chunk, jnp.int32(0))


@functools.partial(
    pl.kernel,
    out_type=jax.ShapeDtypeStruct((E,), jnp.int32),
    mesh=_mesh,
    compiler_params=pltpu.CompilerParams(needs_layout_passes=False),
    scratch_types=[
        pltpu.VMEM((N,), jnp.float32),
        pltpu.VMEM((SCH,), jnp.float32),
        pltpu.VMEM((SCH,), jnp.int32),
        pltpu.VMEM((SCH,), jnp.int32),
        pltpu.VMEM((SCH,), jnp.int32),
    ],
)
def _edge_select(ma_hbm, aff_hbm, row_hbm, col_hbm, out_hbm, ma_v, aff_v, row_v, col_v, out_v):
    c = lax.axis_index("c")
    s = lax.axis_index("s")
    base = (s * 2 + c) * EPW
    pltpu.sync_copy(ma_hbm, ma_v)

    def chunk(k, carry):
        off = base + k * SCH
        pltpu.sync_copy(aff_hbm.at[pl.ds(off, SCH)], aff_v)
        pltpu.sync_copy(row_hbm.at[pl.ds(off, SCH)], row_v)
        pltpu.sync_copy(col_hbm.at[pl.ds(off, SCH)], col_v)

        def grp(g, carry2):
            ds = pl.ds(g * 16, 16)
            rv = row_v[ds]
            cv = col_v[ds]
            mr = plsc.load_gather(ma_v, [rv])
            mc = plsc.load_gather(ma_v, [cv])
            av = aff_v[ds]
            out_v[ds] = jnp.where(av <= jnp.minimum(mr, mc), rv, cv)
            return carry2

        lax.fori_loop(jnp.int32(0), jnp.int32(SCH // 16), grp, jnp.int32(0))
        pltpu.sync_copy(out_v, out_hbm.at[pl.ds(off, SCH)])
        return carry

    lax.fori_loop(jnp.int32(0), jnp.int32(EPW // SCH), chunk, jnp.int32(0))


def _build_csr(src_i, dst_i):
    s_dst, s_src = lax.sort((dst_i, src_i), num_keys=1)
    iota_e = jnp.arange(E, dtype=jnp.int32)
    run_start = jnp.concatenate(
        [jnp.ones((1,), jnp.bool_), s_dst[1:] != s_dst[:-1]]
    )
    seg_start = lax.cummax(jnp.where(run_start, iota_e, 0))
    rank = iota_e - seg_start
    newv = ((rank & 15) == 0).astype(jnp.int32)
    cgrp = jnp.cumsum(newv, dtype=jnp.int32)
    dest = (cgrp - (rank >> 4) - 1) * 16 + rank
    indeg = jnp.zeros((N,), jnp.int32).at[dst_i].add(1)
    nv = (indeg + 15) // 16
    voff_n = jnp.concatenate(
        [jnp.zeros((1,), jnp.int32), jnp.cumsum(nv, dtype=jnp.int32)]
    )
    voff = jnp.concatenate([voff_n, jnp.broadcast_to(voff_n[N], (7,))])
    pad_init = jnp.int32(N) + (jnp.arange(SZ, dtype=jnp.int32) & 1023)
    src_pad = pad_init.at[dest].add(s_src - (jnp.int32(N) + (dest & 1023)))
    return src_pad, voff


def kernel(x, edge_index):
    row, col = edge_index[0], edge_index[1]
    row32 = row.astype(jnp.int32)
    col32 = col.astype(jnp.int32)
    dt = _diff_gather(x.T, row32, col32)
    aff = jnp.linalg.norm(dt.T, axis=1)
    deg = jax.ops.segment_sum(jnp.ones(E, jnp.float32), row, num_segments=N)
    mean_aff = jax.ops.segment_sum(aff, row, num_segments=N) / jnp.maximum(deg, 1.0)
    src_i = _edge_select(mean_aff, aff, row32, col32)
    dst_i = col32

    src_pad, voff = _build_csr(src_i, dst_i)

    sent_vals = jnp.int32(0x40000000) + jnp.arange(SENT, dtype=jnp.int32)

    def cond(st):
        _, i, done = st
        return jnp.logical_and(i < NUM_ITER, jnp.logical_not(done))

    def body(st):
        lab, i, _ = st
        ext = jnp.concatenate([lab, sent_vals])
        new = _lp_step(ext, src_pad, voff)
        return new, i + 1, jnp.array_equal(new, lab)

    labels, _, _ = lax.while_loop(
        cond,
        body,
        (jnp.arange(N, dtype=jnp.int32), jnp.int32(0), jnp.array(False)),
    )

    present = jnp.zeros((N,), jnp.int32).at[labels].add(1)
    rank_incl = jnp.cumsum((present > 0).astype(jnp.int32), dtype=jnp.int32)
    inv = jnp.take(rank_incl, labels) - 1
    num_clusters = rank_incl[N - 1]

    pooled = jax.ops.segment_max(x, inv, num_segments=N)
    nonempty = jnp.arange(N, dtype=jnp.int32) < num_clusters
    return jnp.where(nonempty[:, None], pooled, 0.0)


# speedup vs baseline: 192.4126x; 1.4424x over previous
import functools

import jax
import jax.numpy as jnp
from jax import lax
from jax.experimental import pallas as pl
from jax.experimental.pallas import tpu as pltpu
from jax.experimental.pallas import tpu_sc as plsc

N = 50176
E = 802816
D = 32
NUM_ITER = 70

NW = 32
PER = N // NW
SENT = 1024
NE = N + SENT
GCAP = 4096
CAP = 57344
SZ = E + 16 * N + CAP

_mesh = plsc.VectorSubcoreMesh(core_axis_name="c", subcore_axis_name="s")


@functools.partial(
    pl.kernel,
    out_type=jax.ShapeDtypeStruct((N,), jnp.int32),
    mesh=_mesh,
    compiler_params=pltpu.CompilerParams(needs_layout_passes=False),
    scratch_types=[
        pltpu.VMEM((NE,), jnp.int32),
        pltpu.VMEM((CAP,), jnp.int32),
        pltpu.VMEM((PER + 24,), jnp.int32),
        pltpu.VMEM((GCAP,), jnp.int32),
        pltpu.VMEM((PER,), jnp.int32),
    ],
)
def _lp_step(lab_hbm, src_hbm, voff_hbm, out_hbm, lab_v, src_v, voff_v, g_v, out_v):
    c = lax.axis_index("c")
    s = lax.axis_index("s")
    wid = s * 2 + c
    nbase = wid * PER
    pltpu.sync_copy(lab_hbm, lab_v)
    pltpu.sync_copy(voff_hbm.at[pl.ds(nbase, PER + 8)], voff_v.at[pl.ds(0, PER + 8)])
    voff0 = voff_v[pl.ds(0, 16)][0]
    pltpu.sync_copy(src_hbm.at[pl.ds(voff0 * 16, CAP)], src_v)

    iota16 = lax.iota(jnp.int32, 16)
    rots = [lax.rem(iota16 + jnp.int32(r), jnp.int32(16)) for r in range(16)]
    lane0 = iota16 == 0

    def node_body(d, carry):
        vpair = voff_v[pl.ds(d, 16)]
        v0 = vpair[0] - voff0
        nv = jnp.minimum(vpair[1] - vpair[0], GCAP // 16)
        old = lab_v[pl.ds(nbase + d, 16)][0]

        def gather_one(i, carry2):
            sidx = src_v[pl.ds((v0 + i) * 16, 16)]
            g_v[pl.ds(i * 16, 16)] = plsc.load_gather(lab_v, [sidx])
            return carry2

        lax.fori_loop(jnp.int32(0), nv, gather_one, jnp.int32(0))

        def best_i(i, best):
            gi = g_v[pl.ds(i * 16, 16)]

            def cnt_j(j, cnt):
                gj = g_v[pl.ds(j * 16, 16)]
                for r in range(16):
                    gr = gj.at[rots[r]].get(mode="promise_in_bounds")
                    cnt = cnt + (gi == gr).astype(jnp.int32)
                return cnt

            cnt = lax.fori_loop(jnp.int32(0), nv, cnt_j, jnp.zeros(16, jnp.int32))
            valid = gi < jnp.int32(0x40000000)
            comp = jnp.where(
                valid,
                jnp.left_shift(jnp.minimum(cnt, jnp.int32(0x7FFF)), 16) | gi,
                jnp.int32(0),
            )
            return jnp.maximum(best, jnp.max(comp))

        best = lax.fori_loop(jnp.int32(0), nv, best_i, jnp.int32(0))
        new = jnp.where(best > 0, best & jnp.int32(0xFFFF), old)
        plsc.store_scatter(
            out_v,
            [jnp.broadcast_to(d, (16,))],
            jnp.broadcast_to(new, (16,)),
            mask=lane0,
        )
        return carry

    lax.fori_loop(jnp.int32(0), jnp.int32(PER), node_body, jnp.int32(0))
    pltpu.sync_copy(out_v, out_hbm.at[pl.ds(nbase, PER)])


EPW = E // NW
GCH = 12544
SCH = 1568


@functools.partial(
    pl.kernel,
    out_type=jax.ShapeDtypeStruct((D, E), jnp.float32),
    mesh=_mesh,
    compiler_params=pltpu.CompilerParams(needs_layout_passes=False),
    scratch_types=[
        pltpu.VMEM((N,), jnp.float32),
        pltpu.VMEM((GCH,), jnp.int32),
        pltpu.VMEM((GCH,), jnp.int32),
        pltpu.VMEM((GCH,), jnp.float32),
    ],
)
def _diff_gather(xt_hbm, row_hbm, col_hbm, dt_hbm, tab_v, row_v, col_v, out_v):
    c = lax.axis_index("c")
    s = lax.axis_index("s")
    f = s * 2 + c
    pltpu.sync_copy(xt_hbm.at[f], tab_v)

    def chunk(k, carry):
        off = k * GCH
        pltpu.sync_copy(row_hbm.at[pl.ds(off, GCH)], row_v)
        pltpu.sync_copy(col_hbm.at[pl.ds(off, GCH)], col_v)

        def grp(g, carry2):
            ds = pl.ds(g * 16, 16)
            a = plsc.load_gather(tab_v, [row_v[ds]])
            b = plsc.load_gather(tab_v, [col_v[ds]])
            out_v[ds] = a - b
            return carry2

        lax.fori_loop(jnp.int32(0), jnp.int32(GCH // 16), grp, jnp.int32(0))
        pltpu.sync_copy(out_v, dt_hbm.at[f, pl.ds(off, GCH)])
        return carry

    lax.fori_loop(jnp.int32(0), jnp.int32(E // GCH), chunk, jnp.int32(0))


@functools.partial(
    pl.kernel,
    out_type=(
        jax.ShapeDtypeStruct((E,), jnp.int32),
        jax.ShapeDtypeStruct((E,), jnp.int32),
        jax.ShapeDtypeStruct((NW * N,), jnp.int32),
    ),
    mesh=_mesh,
    compiler_params=pltpu.CompilerParams(needs_layout_passes=False),
    scratch_types=[
        pltpu.VMEM((N,), jnp.float32),
        pltpu.VMEM((N,), jnp.int32),
        pltpu.VMEM((SCH,), jnp.float32),
        pltpu.VMEM((SCH,), jnp.int32),
        pltpu.VMEM((SCH,), jnp.int32),
        pltpu.VMEM((SCH,), jnp.int32),
        pltpu.VMEM((SCH,), jnp.int32),
    ],
)
def _edge_select(ma_hbm, aff_hbm, row_hbm, col_hbm,
                 src_hbm, lrank_hbm, cnt_hbm,
                 ma_v, cnt_v, aff_v, row_v, col_v, src_v, lr_v):
    c = lax.axis_index("c")
    s = lax.axis_index("s")
    w = s * 2 + c
    base = w * EPW
    pltpu.sync_copy(ma_hbm, ma_v)
    zeros16 = jnp.zeros((16,), jnp.int32)

    def zgrp(g, carry):
        cnt_v[pl.ds(g * 16, 16)] = zeros16
        return carry

    lax.fori_loop(jnp.int32(0), jnp.int32(N // 16), zgrp, jnp.int32(0))

    def chunk(k, carry):
        off = base + k * SCH
        pltpu.sync_copy(aff_hbm.at[pl.ds(off, SCH)], aff_v)
        pltpu.sync_copy(row_hbm.at[pl.ds(off, SCH)], row_v)
        pltpu.sync_copy(col_hbm.at[pl.ds(off, SCH)], col_v)

        def grp(g, carry2):
            ds = pl.ds(g * 16, 16)
            rv = row_v[ds]
            cv = col_v[ds]
            mr = plsc.load_gather(ma_v, [rv])
            mc = plsc.load_gather(ma_v, [cv])
            av = aff_v[ds]
            src_v[ds] = jnp.where(av <= jnp.minimum(mr, mc), rv, cv)
            occ, lastm = plsc.scan_count(cv)
            cur = plsc.load_gather(cnt_v, [cv])
            lr_v[ds] = cur + occ - 1
            plsc.store_scatter(cnt_v, [cv], cur + occ, mask=lastm)
            return carry2

        lax.fori_loop(jnp.int32(0), jnp.int32(SCH // 16), grp, jnp.int32(0))
        pltpu.sync_copy(src_v, src_hbm.at[pl.ds(off, SCH)])
        pltpu.sync_copy(lr_v, lrank_hbm.at[pl.ds(off, SCH)])
        return carry

    lax.fori_loop(jnp.int32(0), jnp.int32(EPW // SCH), chunk, jnp.int32(0))
    pltpu.sync_copy(cnt_v, cnt_hbm.at[pl.ds(w * N, N)])


@functools.partial(
    pl.kernel,
    out_type=jax.ShapeDtypeStruct((NW * N,), jnp.int32),
    mesh=_mesh,
    compiler_params=pltpu.CompilerParams(needs_layout_passes=False),
    scratch_types=[
        pltpu.VMEM((NW * PER,), jnp.int32),
        pltpu.VMEM((NW * PER,), jnp.int32),
    ],
)
def _csr_prefix(cnt_hbm, pref_hbm, buf_v, out_v):
    c = lax.axis_index("c")
    s = lax.axis_index("s")
    lo = (s * 2 + c) * PER
    for wp in range(NW):
        pltpu.sync_copy(cnt_hbm.at[pl.ds(wp * N + lo, PER)],
                        buf_v.at[pl.ds(wp * PER, PER)])

    def grp(g, carry):
        acc = jnp.zeros((16,), jnp.int32)
        for wp in range(NW):
            ds = pl.ds(wp * PER + g * 16, 16)
            v = buf_v[ds]
            out_v[ds] = acc
            acc = acc + v
        return carry

    lax.fori_loop(jnp.int32(0), jnp.int32(PER // 16), grp, jnp.int32(0))
    for wp in range(NW):
        pltpu.sync_copy(out_v.at[pl.ds(wp * PER, PER)],
                        pref_hbm.at[pl.ds(wp * N + lo, PER)])


@functools.partial(
    pl.kernel,
    out_type=jax.ShapeDtypeStruct((E,), jnp.int32),
    mesh=_mesh,
    compiler_params=pltpu.CompilerParams(needs_layout_passes=False),
    scratch_types=[
        pltpu.VMEM((N,), jnp.int32),
        pltpu.VMEM((N,), jnp.int32),
        pltpu.VMEM((SCH,), jnp.int32),
        pltpu.VMEM((SCH,), jnp.int32),
        pltpu.VMEM((SCH,), jnp.int32),
    ],
)
def _csr_dest(col_hbm, lrank_hbm, pref_hbm, voff16_hbm, dest_hbm,
              pref_v, voff_v, col_v, lr_v, dest_v):
    c = lax.axis_index("c")
    s = lax.axis_index("s")
    w = s * 2 + c
    base = w * EPW
    pltpu.sync_copy(pref_hbm.at[pl.ds(w * N, N)], pref_v)
    pltpu.sync_copy(voff16_hbm, voff_v)

    def chunk(k, carry):
        off = base + k * SCH
        pltpu.sync_copy(col_hbm.at[pl.ds(off, SCH)], col_v)
        pltpu.sync_copy(lrank_hbm.at[pl.ds(off, SCH)], lr_v)

        def grp(g, carry2):
            ds = pl.ds(g * 16, 16)
            dv = col_v[ds]
            b0 = plsc.load_gather(voff_v, [dv])
            b1 = plsc.load_gather(pref_v, [dv])
            dest_v[ds] = b0 + b1 + lr_v[ds]
            return carry2

        lax.fori_loop(jnp.int32(0), jnp.int32(SCH // 16), grp, jnp.int32(0))
        pltpu.sync_copy(dest_v, dest_hbm.at[pl.ds(off, SCH)])
        return carry

    lax.fori_loop(jnp.int32(0), jnp.int32(EPW // SCH), chunk, jnp.int32(0))


def _build_csr(src_i, dst_i, lrank, cnt):
    indeg = jnp.zeros((N,), jnp.int32).at[dst_i].add(1)
    nv = (indeg + 15) // 16
    voff_n = jnp.concatenate(
        [jnp.zeros((1,), jnp.int32), jnp.cumsum(nv, dtype=jnp.int32)]
    )
    voff = jnp.concatenate([voff_n, jnp.broadcast_to(voff_n[N], (7,))])
    pref = _csr_prefix(cnt)
    dest = _csr_dest(dst_i, lrank, pref, voff_n[:N] * 16)
    pad_init = jnp.int32(N) + (jnp.arange(SZ, dtype=jnp.int32) & 1023)
    src_pad = pad_init.at[dest].add(src_i - (jnp.int32(N) + (dest & 1023)))
    return src_pad, voff


def kernel(x, edge_index):
    row, col = edge_index[0], edge_index[1]
    row32 = row.astype(jnp.int32)
    col32 = col.astype(jnp.int32)
    dt = _diff_gather(x.T, row32, col32)
    aff = jnp.linalg.norm(dt.T, axis=1)
    deg = jax.ops.segment_sum(jnp.ones(E, jnp.float32), row, num_segments=N)
    mean_aff = jax.ops.segment_sum(aff, row, num_segments=N) / jnp.maximum(deg, 1.0)
    src_i, lrank, cnt = _edge_select(mean_aff, aff, row32, col32)
    dst_i = col32

    src_pad, voff = _build_csr(src_i, dst_i, lrank, cnt)

    sent_vals = jnp.int32(0x40000000) + jnp.arange(SENT, dtype=jnp.int32)

    def cond(st):
        _, i, done = st
        return jnp.logical_and(i < NUM_ITER, jnp.logical_not(done))

    def body(st):
        lab, i, _ = st
        ext = jnp.concatenate([lab, sent_vals])
        new = _lp_step(ext, src_pad, voff)
        return new, i + 1, jnp.array_equal(new, lab)

    labels, _, _ = lax.while_loop(
        cond,
        body,
        (jnp.arange(N, dtype=jnp.int32), jnp.int32(0), jnp.array(False)),
    )

    present = jnp.zeros((N,), jnp.int32).at[labels].add(1)
    rank_incl = jnp.cumsum((present > 0).astype(jnp.int32), dtype=jnp.int32)
    inv = jnp.take(rank_incl, labels) - 1
    num_clusters = rank_incl[N - 1]

    pooled = jax.ops.segment_max(x, inv, num_segments=N)
    nonempty = jnp.arange(N, dtype=jnp.int32) < num_clusters
    return jnp.where(nonempty[:, None], pooled, 0.0)
